# Initial kernel scaffold; baseline (speedup 1.0000x reference)
import jax
import jax.numpy as jnp
from jax.experimental import pallas as pl


def kernel(x_txt, x_img, x_struct, edge_index, params):
    raise NotImplementedError("write your pallas kernel here")


# speedup vs baseline: 8.3681x; 8.3681x over previous
import functools

import jax
import jax.numpy as jnp
from jax import lax
from jax.experimental import pallas as pl
from jax.experimental.pallas import tpu as pltpu
from jax.experimental.pallas import tpu_sc as plsc

N_USERS = 2000
N_ITEMS = 8000
N = N_USERS + N_ITEMS
NP = 10240
D = 128
F = 3 * D
FQ = F // 4
E = 320000
E2 = 2 * E
EP = 643072
CH = 128
EPW32 = EP // 32
EPW16 = EP // 16
ROWS_PER_TILE = NP // 16

def _sc_mesh():
    return plsc.VectorSubcoreMesh(core_axis_name="c", subcore_axis_name="s")


_SC_PARAMS = pltpu.CompilerParams(use_tc_tiling_on_sc=False)


@functools.cache
def _deg_kernel_fn():
    return functools.partial(
        pl.kernel,
        out_type=jax.ShapeDtypeStruct((2 * NP, 16), jnp.float32),
        mesh=_sc_mesh(),
        scratch_types=[
            pltpu.VMEM((CH,), jnp.int32),
            pltpu.VMEM((CH, 16), jnp.float32),
            pltpu.VMEM_SHARED((NP, 16), jnp.float32),
        ],
        compiler_params=_SC_PARAMS,
    )(_deg_body)


def _deg_body(srcs_hbm, zeros1_hbm, ones1_hbm, deg_hbm, ibuf, obuf, dacc):
    c = lax.axis_index("c")
    s = lax.axis_index("s")
    wid = s * 2 + c

    pltpu.sync_copy(zeros1_hbm, obuf)
    tb = s * ROWS_PER_TILE
    for k in range(ROWS_PER_TILE // CH):
        pltpu.sync_copy(obuf, dacc.at[pl.ds(tb + k * CH, CH)])
    plsc.subcore_barrier()

    pltpu.sync_copy(ones1_hbm, obuf)

    def edge_body(j, _):
        be = wid * EPW32 + j * CH
        pltpu.sync_copy(srcs_hbm.at[pl.ds(be, CH)], ibuf)
        pltpu.sync_copy(obuf, dacc.at[ibuf], add=True)
        return 0

    lax.fori_loop(0, EPW32 // CH, edge_body, 0)
    plsc.subcore_barrier()

    for k in range(ROWS_PER_TILE // CH):
        pltpu.sync_copy(dacc.at[pl.ds(tb + k * CH, CH)], obuf)
        pltpu.sync_copy(obuf, deg_hbm.at[pl.ds(c * NP + tb + k * CH, CH)])


@functools.cache
def _prop_kernel_fn():
    return functools.partial(
        pl.kernel,
        out_type=jax.ShapeDtypeStruct((4 * NP, FQ), jnp.float32),
        mesh=_sc_mesh(),
        scratch_types=[
            pltpu.VMEM((CH,), jnp.int32),
            pltpu.VMEM((CH,), jnp.int32),
            pltpu.VMEM((CH, FQ), jnp.float32),
            pltpu.VMEM_SHARED((NP, FQ), jnp.float32),
            pltpu.SemaphoreType.DMA,
        ],
        compiler_params=_SC_PARAMS,
    )(_prop_body)


def _prop_body(y_hbm, srcs4_hbm, dsts_hbm, zrows_hbm, out_hbm,
               sbuf, dbuf, rows, accum, sem):
    c = lax.axis_index("c")
    s = lax.axis_index("s")
    tb = s * ROWS_PER_TILE

    for h in range(2):
        q = 2 * h + c
        pltpu.sync_copy(zrows_hbm, rows)
        for k in range(ROWS_PER_TILE // CH):
            pltpu.sync_copy(rows, accum.at[pl.ds(tb + k * CH, CH)])
        plsc.subcore_barrier()

        def edge_body(j, _):
            be = s * EPW16 + j * CH
            pltpu.sync_copy(srcs4_hbm.at[q, pl.ds(be, CH)], sbuf)
            pltpu.sync_copy(dsts_hbm.at[pl.ds(be, CH)], dbuf)
            pltpu.async_copy(y_hbm.at[sbuf], rows, sem).wait()
            pltpu.sync_copy(rows, accum.at[dbuf], add=True)
            return 0

        lax.fori_loop(0, EPW16 // CH, edge_body, 0)
        plsc.subcore_barrier()

        for k in range(ROWS_PER_TILE // CH):
            pltpu.sync_copy(accum.at[pl.ds(tb + k * CH, CH)], rows)
            pltpu.sync_copy(rows, out_hbm.at[pl.ds(q * NP + tb + k * CH, CH)])
        plsc.subcore_barrier()


def _proj_body(x_ref, w_ref, b_ref, o_ref):
    o_ref[...] = (
        jnp.dot(x_ref[...], w_ref[...], preferred_element_type=jnp.float32)
        + b_ref[...]
    )


def _projection(xcat, wcat, bcat):
    mblk = 1000
    return pl.pallas_call(
        _proj_body,
        grid=(N_ITEMS // mblk,),
        in_specs=[
            pl.BlockSpec((mblk, 3 * (384 + 512 + 64) // 3), lambda i: (i, 0)),
            pl.BlockSpec((384 + 512 + 64, F), lambda i: (0, 0)),
            pl.BlockSpec((1, F), lambda i: (0, 0)),
        ],
        out_specs=pl.BlockSpec((mblk, F), lambda i: (i, 0)),
        out_shape=jax.ShapeDtypeStruct((N_ITEMS, F), jnp.float32),
    )(xcat, wcat, bcat)


def _dm_of(d_ref):
    deg = d_ref[..., 0:1] + d_ref[..., 1:2]
    return lax.rsqrt(jnp.clip(deg, 1.0, None))


def _scale_body(x_ref, d_ref, o_ref):
    o_ref[...] = x_ref[...] * _dm_of(d_ref)


def _scale_sq_body(x_ref, d_ref, o_ref):
    deg = d_ref[..., 0:1] + d_ref[..., 1:2]
    o_ref[...] = x_ref[...] / jnp.clip(deg, 1.0, None)


def _rowscale(x, degp, squared):
    mblk = 1280
    return pl.pallas_call(
        _scale_sq_body if squared else _scale_body,
        grid=(NP // mblk,),
        in_specs=[
            pl.BlockSpec((mblk, F), lambda i: (i, 0)),
            pl.BlockSpec((mblk, 2), lambda i: (i, 0)),
        ],
        out_specs=pl.BlockSpec((mblk, F), lambda i: (i, 0)),
        out_shape=jax.ShapeDtypeStruct((NP, F), jnp.float32),
    )(x, degp)


def _final_body(x0_ref, s1_ref, s2_ref, d_ref,
                w1u_ref, b1u_ref, w2u_ref, b2u_ref,
                w1i_ref, b1i_ref, w2i_ref, b2i_ref, o_ref):
    dm = _dm_of(d_ref)
    z = (x0_ref[...] + dm * (s1_ref[...] + s2_ref[...])) * (1.0 / 3.0)
    is_user = pl.program_id(0) < (N_USERS // 400)
    w1 = jnp.where(is_user, w1u_ref[...], w1i_ref[...])
    b1 = jnp.where(is_user, b1u_ref[...], b1i_ref[...])
    w2 = jnp.where(is_user, w2u_ref[...], w2i_ref[...])
    b2 = jnp.where(is_user, b2u_ref[...], b2i_ref[...])
    h = jnp.maximum(jnp.dot(z, w1, preferred_element_type=jnp.float32) + b1, 0.0)
    logits = jnp.dot(h, w2, preferred_element_type=jnp.float32) + b2
    m = jnp.max(logits, axis=-1, keepdims=True)
    ex = jnp.exp(logits - m)
    a = ex / jnp.sum(ex, axis=-1, keepdims=True)
    o_ref[...] = (a[:, 0:1] * z[:, 0:D]
                  + a[:, 1:2] * z[:, D:2 * D]
                  + a[:, 2:3] * z[:, 2 * D:3 * D])


def _final(x0, s1, s2, degp, w1u, b1u, w2u, b2u, w1i, b1i, w2i, b2i):
    mblk = 400
    row = lambda i: (i, 0)
    fixed = lambda i: (0, 0)
    return pl.pallas_call(
        _final_body,
        grid=(N // mblk,),
        in_specs=[
            pl.BlockSpec((mblk, F), row),
            pl.BlockSpec((mblk, F), row),
            pl.BlockSpec((mblk, F), row),
            pl.BlockSpec((mblk, 2), row),
            pl.BlockSpec((F, D), fixed),
            pl.BlockSpec((1, D), fixed),
            pl.BlockSpec((D, D), fixed),
            pl.BlockSpec((1, D), fixed),
            pl.BlockSpec((F, D), fixed),
            pl.BlockSpec((1, D), fixed),
            pl.BlockSpec((D, D), fixed),
            pl.BlockSpec((1, D), fixed),
        ],
        out_specs=pl.BlockSpec((mblk, D), row),
        out_shape=jax.ShapeDtypeStruct((N, D), jnp.float32),
    )(x0, s1, s2, degp, w1u, b1u, w2u, b2u, w1i, b1i, w2i, b2i)


def _to_sc_layout(y):
    return y.reshape(NP, 4, FQ).transpose(1, 0, 2).reshape(4 * NP, FQ)


def _from_sc_layout(s):
    return s.reshape(4, NP, FQ).transpose(1, 0, 2).reshape(NP, F)


def _pad_mlp(p):
    w2 = jnp.zeros((D, D), jnp.float32).at[:, :3].set(p["W2"])
    b2 = jnp.full((1, D), -1e30, jnp.float32).at[0, :3].set(p["b2"])
    return (p["W1"], p["b1"].reshape(1, D), w2, b2)


def kernel(x_txt, x_img, x_struct, edge_index, params):
    e = edge_index.astype(jnp.int32)
    pad_s = jnp.full((EP - E2,), N, jnp.int32)
    pad_d = jnp.full((EP - E2,), NP - 1, jnp.int32)
    srcs = jnp.concatenate([e[0], e[1], pad_s])
    dsts = jnp.concatenate([e[1], e[0], pad_d])
    srcs4 = jnp.stack([srcs + q * NP for q in range(4)])
    zeros1 = jnp.zeros((CH, 16), jnp.float32)
    ones1 = jnp.ones((CH, 16), jnp.float32)
    zrows = jnp.zeros((CH, FQ), jnp.float32)

    degp = _deg_kernel_fn()(srcs, zeros1, ones1)[:, 0].reshape(2, NP).transpose(1, 0)

    xcat = jnp.concatenate([x_txt, x_img, x_struct], axis=1)
    bp = [params["txt"], params["img"], params["st"]]
    wcat = jnp.concatenate(
        [jnp.concatenate([p["Wt"], p["Wi"], p["Ws"]], axis=0) for p in bp], axis=1)
    bcat = jnp.concatenate([p["bt"] + p["bi"] + p["bs"] for p in bp]).reshape(1, F)
    i_feat = _projection(xcat, wcat, bcat)

    user = jnp.concatenate([p["user"] for p in bp], axis=1)
    x0 = jnp.concatenate(
        [user, i_feat, jnp.zeros((NP - N, F), jnp.float32)], axis=0)

    y0 = _rowscale(x0, degp, squared=False)
    prop = _prop_kernel_fn()
    s1 = _from_sc_layout(prop(_to_sc_layout(y0), srcs4, dsts, zrows))
    y1 = _rowscale(s1, degp, squared=True)
    s2 = _from_sc_layout(prop(_to_sc_layout(y1), srcs4, dsts, zrows))

    u = _pad_mlp(params["attn_u"])
    i = _pad_mlp(params["attn_i"])
    return _final(x0, s1, s2, degp, *u, *i)


# speedup vs baseline: 14.3437x; 1.7141x over previous
import functools

import jax
import jax.numpy as jnp
from jax import lax
from jax.experimental import pallas as pl
from jax.experimental.pallas import tpu as pltpu
from jax.experimental.pallas import tpu_sc as plsc

N_USERS = 2000
N_ITEMS = 8000
N = N_USERS + N_ITEMS
NP = 10240
D = 128
F = 3 * D
FQ = F // 4
E = 320000
E2 = 2 * E
EP = 643072
CH = 128
EPW32 = EP // 32
EPW16 = EP // 16
ROWS_PER_TILE = NP // 16

def _sc_mesh():
    return plsc.VectorSubcoreMesh(core_axis_name="c", subcore_axis_name="s")


_SC_PARAMS = pltpu.CompilerParams(use_tc_tiling_on_sc=False)


@functools.cache
def _deg_kernel_fn():
    return functools.partial(
        pl.kernel,
        out_type=jax.ShapeDtypeStruct((2 * NP, 16), jnp.float32),
        mesh=_sc_mesh(),
        scratch_types=[
            pltpu.VMEM((CH,), jnp.int32),
            pltpu.VMEM((CH, 16), jnp.float32),
            pltpu.VMEM_SHARED((NP, 16), jnp.float32),
        ],
        compiler_params=_SC_PARAMS,
    )(_deg_body)


def _deg_body(srcs_hbm, zeros1_hbm, ones1_hbm, deg_hbm, ibuf, obuf, dacc):
    c = lax.axis_index("c")
    s = lax.axis_index("s")
    wid = s * 2 + c

    pltpu.sync_copy(zeros1_hbm, obuf)
    tb = s * ROWS_PER_TILE
    for k in range(ROWS_PER_TILE // CH):
        pltpu.sync_copy(obuf, dacc.at[pl.ds(tb + k * CH, CH)])
    plsc.subcore_barrier()

    pltpu.sync_copy(ones1_hbm, obuf)

    def edge_body(j, _):
        be = wid * EPW32 + j * CH
        pltpu.sync_copy(srcs_hbm.at[pl.ds(be, CH)], ibuf)
        pltpu.sync_copy(obuf, dacc.at[ibuf], add=True)
        return 0

    lax.fori_loop(0, EPW32 // CH, edge_body, 0)
    plsc.subcore_barrier()

    for k in range(ROWS_PER_TILE // CH):
        pltpu.sync_copy(dacc.at[pl.ds(tb + k * CH, CH)], obuf)
        pltpu.sync_copy(obuf, deg_hbm.at[pl.ds(c * NP + tb + k * CH, CH)])


CHUNKS = EPW16 // CH


@functools.cache
def _prop_kernel_fn():
    return functools.partial(
        pl.kernel,
        out_type=jax.ShapeDtypeStruct((4 * NP, FQ), jnp.float32),
        mesh=_sc_mesh(),
        scratch_types=[
            pltpu.VMEM((2, CH), jnp.int32),
            pltpu.VMEM((2, CH), jnp.int32),
            pltpu.VMEM((CH, FQ), jnp.float32),
            pltpu.VMEM((CH, FQ), jnp.float32),
            pltpu.VMEM_SHARED((NP, FQ), jnp.float32),
            pltpu.SemaphoreType.DMA,
            pltpu.SemaphoreType.DMA,
            pltpu.SemaphoreType.DMA,
            pltpu.SemaphoreType.DMA,
        ],
        compiler_params=_SC_PARAMS,
    )(_prop_body)


def _prop_body(y_hbm, esd_hbm, zrows_hbm, out_hbm,
               ia, ib, ra, rb, accum, sga, sgb, ssa, ssb):
    c = lax.axis_index("c")
    s = lax.axis_index("s")
    tb = s * ROWS_PER_TILE

    for h in range(2):
        q = 2 * h + c
        pltpu.sync_copy(zrows_hbm, ra)
        for k in range(ROWS_PER_TILE // CH):
            pltpu.sync_copy(ra, accum.at[pl.ds(tb + k * CH, CH)])
        plsc.subcore_barrier()

        def edge_body(j, _):
            g = s * CHUNKS + 2 * j

            @pl.when(j > 0)
            def _():
                pltpu.make_async_copy(ra, accum.at[ia.at[1]], ssa).wait()

            pltpu.sync_copy(esd_hbm.at[q, g], ia)
            ga = pltpu.async_copy(y_hbm.at[ia.at[0]], ra, sga)

            @pl.when(j > 0)
            def _():
                pltpu.make_async_copy(rb, accum.at[ib.at[1]], ssb).wait()

            pltpu.sync_copy(esd_hbm.at[q, g + 1], ib)
            gb = pltpu.async_copy(y_hbm.at[ib.at[0]], rb, sgb)
            ga.wait()
            pltpu.async_copy(ra, accum.at[ia.at[1]], ssa, add=True)
            gb.wait()
            pltpu.async_copy(rb, accum.at[ib.at[1]], ssb, add=True)
            return 0

        lax.fori_loop(0, CHUNKS // 2, edge_body, 0)
        pltpu.make_async_copy(ra, accum.at[ia.at[1]], ssa).wait()
        pltpu.make_async_copy(rb, accum.at[ib.at[1]], ssb).wait()
        plsc.subcore_barrier()

        for k in range(ROWS_PER_TILE // CH):
            pltpu.sync_copy(accum.at[pl.ds(tb + k * CH, CH)], ra)
            pltpu.sync_copy(ra, out_hbm.at[pl.ds(q * NP + tb + k * CH, CH)])
        plsc.subcore_barrier()


def _proj_body(x_ref, w_ref, b_ref, o_ref):
    o_ref[...] = (
        jnp.dot(x_ref[...], w_ref[...], preferred_element_type=jnp.float32)
        + b_ref[...]
    )


def _projection(xcat, wcat, bcat):
    mblk = 1000
    return pl.pallas_call(
        _proj_body,
        grid=(N_ITEMS // mblk,),
        in_specs=[
            pl.BlockSpec((mblk, 3 * (384 + 512 + 64) // 3), lambda i: (i, 0)),
            pl.BlockSpec((384 + 512 + 64, F), lambda i: (0, 0)),
            pl.BlockSpec((1, F), lambda i: (0, 0)),
        ],
        out_specs=pl.BlockSpec((mblk, F), lambda i: (i, 0)),
        out_shape=jax.ShapeDtypeStruct((N_ITEMS, F), jnp.float32),
    )(xcat, wcat, bcat)


def _dm_of(d_ref):
    deg = d_ref[..., 0:1] + d_ref[..., 1:2]
    return lax.rsqrt(jnp.clip(deg, 1.0, None))


def _scale_body(x_ref, d_ref, o_ref):
    o_ref[...] = x_ref[...] * _dm_of(d_ref)


def _scale_sq_body(x_ref, d_ref, o_ref):
    deg = d_ref[..., 0:1] + d_ref[..., 1:2]
    o_ref[...] = x_ref[...] / jnp.clip(deg, 1.0, None)


def _rowscale(x, degp, squared):
    mblk = 1280
    return pl.pallas_call(
        _scale_sq_body if squared else _scale_body,
        grid=(NP // mblk,),
        in_specs=[
            pl.BlockSpec((mblk, F), lambda i: (i, 0)),
            pl.BlockSpec((mblk, 2), lambda i: (i, 0)),
        ],
        out_specs=pl.BlockSpec((mblk, F), lambda i: (i, 0)),
        out_shape=jax.ShapeDtypeStruct((NP, F), jnp.float32),
    )(x, degp)


def _final_body(x0_ref, s1_ref, s2_ref, d_ref,
                w1u_ref, b1u_ref, w2u_ref, b2u_ref,
                w1i_ref, b1i_ref, w2i_ref, b2i_ref, o_ref):
    dm = _dm_of(d_ref)
    z = (x0_ref[...] + dm * (s1_ref[...] + s2_ref[...])) * (1.0 / 3.0)
    is_user = pl.program_id(0) < (N_USERS // 400)
    w1 = jnp.where(is_user, w1u_ref[...], w1i_ref[...])
    b1 = jnp.where(is_user, b1u_ref[...], b1i_ref[...])
    w2 = jnp.where(is_user, w2u_ref[...], w2i_ref[...])
    b2 = jnp.where(is_user, b2u_ref[...], b2i_ref[...])
    h = jnp.maximum(jnp.dot(z, w1, preferred_element_type=jnp.float32) + b1, 0.0)
    logits = jnp.dot(h, w2, preferred_element_type=jnp.float32) + b2
    m = jnp.max(logits, axis=-1, keepdims=True)
    ex = jnp.exp(logits - m)
    a = ex / jnp.sum(ex, axis=-1, keepdims=True)
    o_ref[...] = (a[:, 0:1] * z[:, 0:D]
                  + a[:, 1:2] * z[:, D:2 * D]
                  + a[:, 2:3] * z[:, 2 * D:3 * D])


def _final(x0, s1, s2, degp, w1u, b1u, w2u, b2u, w1i, b1i, w2i, b2i):
    mblk = 400
    row = lambda i: (i, 0)
    fixed = lambda i: (0, 0)
    return pl.pallas_call(
        _final_body,
        grid=(N // mblk,),
        in_specs=[
            pl.BlockSpec((mblk, F), row),
            pl.BlockSpec((mblk, F), row),
            pl.BlockSpec((mblk, F), row),
            pl.BlockSpec((mblk, 2), row),
            pl.BlockSpec((F, D), fixed),
            pl.BlockSpec((1, D), fixed),
            pl.BlockSpec((D, D), fixed),
            pl.BlockSpec((1, D), fixed),
            pl.BlockSpec((F, D), fixed),
            pl.BlockSpec((1, D), fixed),
            pl.BlockSpec((D, D), fixed),
            pl.BlockSpec((1, D), fixed),
        ],
        out_specs=pl.BlockSpec((mblk, D), row),
        out_shape=jax.ShapeDtypeStruct((N, D), jnp.float32),
    )(x0, s1, s2, degp, w1u, b1u, w2u, b2u, w1i, b1i, w2i, b2i)


def _to_sc_layout(y):
    return y.reshape(NP, 4, FQ).transpose(1, 0, 2).reshape(4 * NP, FQ)


def _from_sc_layout(s):
    return s.reshape(4, NP, FQ).transpose(1, 0, 2).reshape(NP, F)


def _pad_mlp(p):
    w2 = jnp.zeros((D, D), jnp.float32).at[:, :3].set(p["W2"])
    b2 = jnp.full((1, D), -1e30, jnp.float32).at[0, :3].set(p["b2"])
    return (p["W1"], p["b1"].reshape(1, D), w2, b2)


def kernel(x_txt, x_img, x_struct, edge_index, params):
    e = edge_index.astype(jnp.int32)
    pad_s = jnp.full((EP - E2,), N, jnp.int32)
    pad_d = jnp.full((EP - E2,), NP - 1, jnp.int32)
    srcs = jnp.concatenate([e[0], e[1], pad_s])
    dsts = jnp.concatenate([e[1], e[0], pad_d])
    srcs_r = srcs.reshape(EP // CH, CH)
    dsts_r = dsts.reshape(EP // CH, CH)
    esd = jnp.stack(
        [jnp.stack([srcs_r + q * NP, dsts_r], axis=1) for q in range(4)])
    zeros1 = jnp.zeros((CH, 16), jnp.float32)
    ones1 = jnp.ones((CH, 16), jnp.float32)
    zrows = jnp.zeros((CH, FQ), jnp.float32)

    degp = _deg_kernel_fn()(srcs, zeros1, ones1)[:, 0].reshape(2, NP).transpose(1, 0)

    xcat = jnp.concatenate([x_txt, x_img, x_struct], axis=1)
    bp = [params["txt"], params["img"], params["st"]]
    wcat = jnp.concatenate(
        [jnp.concatenate([p["Wt"], p["Wi"], p["Ws"]], axis=0) for p in bp], axis=1)
    bcat = jnp.concatenate([p["bt"] + p["bi"] + p["bs"] for p in bp]).reshape(1, F)
    i_feat = _projection(xcat, wcat, bcat)

    user = jnp.concatenate([p["user"] for p in bp], axis=1)
    x0 = jnp.concatenate(
        [user, i_feat, jnp.zeros((NP - N, F), jnp.float32)], axis=0)

    y0 = _rowscale(x0, degp, squared=False)
    prop = _prop_kernel_fn()
    s1 = _from_sc_layout(prop(_to_sc_layout(y0), esd, zrows))
    y1 = _rowscale(s1, degp, squared=True)
    s2 = _from_sc_layout(prop(_to_sc_layout(y1), esd, zrows))

    u = _pad_mlp(params["attn_u"])
    i = _pad_mlp(params["attn_i"])
    return _final(x0, s1, s2, degp, *u, *i)
